# baseline (speedup 1.0000x reference)
import jax
import jax.numpy as jnp
from jax.experimental import pallas as pl
from jax.experimental.pallas import tpu as pltpu

N = 100000
C = 128
T = 4
EPS = 1e-05
BLK = 2000
NSTEPS = N // BLK


def _fused_body(w, b, sb, pq, pv, nb, osb, opq, opv, onb, acc, ss):
    p = pl.program_id(0)
    i = pl.program_id(1)

    @pl.when((p == 0) & (i == 0))
    def _init():
        acc[...] = jnp.zeros_like(acc)

    @pl.when(p == 0)
    def _stats():
        srows, qrows = [], []
        for ref in (sb, pq, pv, nb):
            x = ref[...]
            xr = x.reshape(BLK // 8, 8, C)
            srows.append(jnp.sum(xr, axis=0))
            qrows.append(jnp.sum(xr * xr, axis=0))
        acc[...] += jnp.stack(srows + qrows, axis=0)

    @pl.when((p == 1) & (i == 0))
    def _scale():
        tot = jnp.sum(acc[...], axis=1)
        inv_n = 1.0 / N
        mean = tot[:T, :] * inv_n
        var = tot[T:, :] * inv_n - mean * mean
        inv_std = jax.lax.rsqrt(jnp.clip(var, EPS, None))
        scale = w[...] * inv_std
        shift = b[...] - mean * scale
        ss[...] = jnp.concatenate([scale, shift], axis=0)

    @pl.when(p == 1)
    def _norm():
        for t, (ref, oref) in enumerate(((sb, osb), (pq, opq), (pv, opv), (nb, onb))):
            oref[...] = ref[...] * ss[t:t + 1, :] + ss[T + t:T + t + 1, :]


@jax.jit
def kernel(SB, PQ, PV, NB, weight, bias):
    data_spec = pl.BlockSpec((BLK, C), lambda p, i: (i, 0))
    out_spec = pl.BlockSpec((BLK, C), lambda p, i: (jnp.where(p == 0, 0, i), 0))
    wb_spec = pl.BlockSpec((T, C), lambda p, i: (0, 0))
    outs = pl.pallas_call(
        _fused_body,
        grid=(2, NSTEPS),
        in_specs=[wb_spec, wb_spec] + [data_spec] * 4,
        out_specs=[out_spec] * 4,
        out_shape=[jax.ShapeDtypeStruct((N, C), jnp.float32)] * 4,
        scratch_shapes=[
            pltpu.VMEM((2 * T, 8, C), jnp.float32),
            pltpu.VMEM((2 * T, C), jnp.float32),
        ],
    )(weight, bias, SB, PQ, PV, NB)
    return tuple(outs)


# speedup vs baseline: 1.0855x; 1.0855x over previous
import jax
import jax.numpy as jnp
from jax.experimental import pallas as pl
from jax.experimental.pallas import tpu as pltpu

N = 100000
C = 128
T = 4
EPS = 1e-05
BLK = 5000
NSTEPS = N // BLK


def _fused_body(w, b, sb, pq, pv, nb, osb, opq, opv, onb, acc, ss):
    p = pl.program_id(0)
    i = pl.program_id(1)

    @pl.when((p == 0) & (i == 0))
    def _init():
        acc[...] = jnp.zeros_like(acc)

    @pl.when(p == 0)
    def _stats():
        srows, qrows = [], []
        for ref in (sb, pq, pv, nb):
            x = ref[...]
            xr = x.reshape(BLK // 8, 8, C)
            srows.append(jnp.sum(xr, axis=0))
            qrows.append(jnp.sum(xr * xr, axis=0))
        acc[...] += jnp.stack(srows + qrows, axis=0)

    @pl.when((p == 1) & (i == 0))
    def _scale():
        tot = jnp.sum(acc[...], axis=1)
        inv_n = 1.0 / N
        mean = tot[:T, :] * inv_n
        var = tot[T:, :] * inv_n - mean * mean
        inv_std = jax.lax.rsqrt(jnp.clip(var, EPS, None))
        scale = w[...] * inv_std
        shift = b[...] - mean * scale
        ss[...] = jnp.concatenate([scale, shift], axis=0)

    @pl.when(p == 1)
    def _norm():
        for t, (ref, oref) in enumerate(((sb, osb), (pq, opq), (pv, opv), (nb, onb))):
            oref[...] = ref[...] * ss[t:t + 1, :] + ss[T + t:T + t + 1, :]


@jax.jit
def kernel(SB, PQ, PV, NB, weight, bias):
    data_spec = pl.BlockSpec((BLK, C), lambda p, i: (i, 0))
    out_spec = pl.BlockSpec((BLK, C), lambda p, i: (jnp.where(p == 0, 0, i), 0))
    wb_spec = pl.BlockSpec((T, C), lambda p, i: (0, 0))
    outs = pl.pallas_call(
        _fused_body,
        grid=(2, NSTEPS),
        in_specs=[wb_spec, wb_spec] + [data_spec] * 4,
        out_specs=[out_spec] * 4,
        out_shape=[jax.ShapeDtypeStruct((N, C), jnp.float32)] * 4,
        scratch_shapes=[
            pltpu.VMEM((2 * T, 8, C), jnp.float32),
            pltpu.VMEM((2 * T, C), jnp.float32),
        ],
    )(weight, bias, SB, PQ, PV, NB)
    return tuple(outs)
